# baseline (speedup 1.0000x reference)
import functools

import jax
import jax.numpy as jnp
from jax import lax
from jax.experimental import pallas as pl
from jax.experimental.pallas import tpu as pltpu
from jax.experimental.pallas import tpu_sc as plsc

NUM_FRAMES = 64
RESOLUTION = 64
OCC_THRE = 0.3

N = 1048576
G_ROWS = NUM_FRAMES * RESOLUTION * RESOLUTION
G_PAD = G_ROWS * 128

TC_STEPS = 32
BP = N // TC_STEPS
BR = G_ROWS // TC_STEPS
BO = BR * 128

NC = 2
NS = 16
NW = NC * NS
PPW = N // NW
CHUNK = 16384
NCHUNK = PPW // CHUNK

_INV63 = 1.0 / 63.0


def _tc_body(g2_ref, ptsT_ref, ts_ref, gpad_ref, idx_ref):
    x = g2_ref[...]
    gpad_ref[...] = jnp.pad(x, ((0, 0), (0, 64))).reshape(BO)

    def vox(row):
        g = (ptsT_ref[row, :] * RESOLUTION).astype(jnp.int32)
        return jnp.clip(g, 0, RESOLUTION - 1)

    gx = vox(0)
    gy = vox(1)
    gz = vox(2)
    t = ts_ref[...]
    i0 = jnp.clip((t * (NUM_FRAMES - 1)).astype(jnp.int32) + 1,
                  1, NUM_FRAMES - 1)
    left = (i0 - 1).astype(jnp.float32) * _INV63
    right = i0.astype(jnp.float32) * _INV63
    fidx = jnp.where((t - left) <= (right - t), i0 - 1, i0)
    idx_ref[...] = ((fidx << 19) | (gx << 13) | (gy << 7) | gz)


def _sc_body(gpad_hbm, idx_hbm, vals_hbm, idx_v0, idx_v1, vals_v0, vals_v1,
             sem_in0, sem_in1, sem_out0, sem_out1, sem_g):
    wid = lax.axis_index("s") * NC + lax.axis_index("c")
    base = wid * PPW
    sem_in = (sem_in0, sem_in1)
    sem_out = (sem_out0, sem_out1)
    idx_v = (idx_v0, idx_v1)
    vals_v = (vals_v0, vals_v1)

    def in_copy(c, b):
        return pltpu.make_async_copy(
            idx_hbm.at[pl.ds(base + c * CHUNK, CHUNK)], idx_v[b],
            sem_in[b])

    def out_copy(c, b):
        return pltpu.make_async_copy(
            vals_v[b], vals_hbm.at[pl.ds(base + c * CHUNK, CHUNK)],
            sem_out[b])

    in_copy(0, 0).start()
    for c in range(NCHUNK):
        b = c & 1
        in_copy(c, b).wait()
        if c + 1 < NCHUNK:
            in_copy(c + 1, 1 - b).start()
        if c >= 2:
            out_copy(c - 2, b).wait()
        pltpu.async_copy(gpad_hbm.at[idx_v[b]], vals_v[b], sem_g).wait()
        out_copy(c, b).start()
    out_copy(NCHUNK - 2, 0).wait()
    out_copy(NCHUNK - 1, 1).wait()


@jax.jit
def kernel(pts, ts, ts_keyframes, occ_val_grid):
    ptsT = pts.T
    g2 = occ_val_grid.reshape(G_ROWS, RESOLUTION)

    gpad, idx = pl.pallas_call(
        _tc_body,
        grid=(TC_STEPS,),
        in_specs=[
            pl.BlockSpec((BR, RESOLUTION), lambda i: (i, 0)),
            pl.BlockSpec((3, BP), lambda i: (0, i)),
            pl.BlockSpec((BP,), lambda i: (i,)),
        ],
        out_specs=[
            pl.BlockSpec((BO,), lambda i: (i,)),
            pl.BlockSpec((BP,), lambda i: (i,)),
        ],
        out_shape=[
            jax.ShapeDtypeStruct((G_PAD,), jnp.float32),
            jax.ShapeDtypeStruct((N,), jnp.int32),
        ],
        compiler_params=pltpu.CompilerParams(
            dimension_semantics=("arbitrary",),
        ),
    )(g2, ptsT, ts)

    mesh = plsc.VectorSubcoreMesh(core_axis_name="c", subcore_axis_name="s")
    fn = pl.kernel(
        _sc_body,
        mesh=mesh,
        compiler_params=pltpu.CompilerParams(needs_layout_passes=False),
        out_type=jax.ShapeDtypeStruct((N,), jnp.float32),
        scratch_types=[
            pltpu.VMEM((CHUNK,), jnp.int32),
            pltpu.VMEM((CHUNK,), jnp.int32),
            pltpu.VMEM((CHUNK,), jnp.float32),
            pltpu.VMEM((CHUNK,), jnp.float32),
            pltpu.SemaphoreType.DMA,
            pltpu.SemaphoreType.DMA,
            pltpu.SemaphoreType.DMA,
            pltpu.SemaphoreType.DMA,
            pltpu.SemaphoreType.DMA,
        ],
    )
    vals = fn(gpad, idx)
    return (vals, vals > OCC_THRE)


# speedup vs baseline: 1.1305x; 1.1305x over previous
import functools

import jax
import jax.numpy as jnp
from jax import lax
from jax.experimental import pallas as pl
from jax.experimental.pallas import tpu as pltpu
from jax.experimental.pallas import tpu_sc as plsc

NUM_FRAMES = 64
RESOLUTION = 64
OCC_THRE = 0.3

N = 1048576
G_ROWS = NUM_FRAMES * RESOLUTION * RESOLUTION
G_ELEMS = G_ROWS * RESOLUTION

TC_STEPS = 32
BP = N // TC_STEPS
BR = G_ROWS // TC_STEPS
H = BR // 2
BO = BR * RESOLUTION

NC = 2
NS = 16
NW = NC * NS
PPW = N // NW
CHUNK = 16384
NCHUNK = PPW // CHUNK

_INV63 = 1.0 / 63.0


def _tc_body(g2_ref, ptsT_ref, ts_ref, gpk_ref, idx_ref):
    x = g2_ref[...]
    y = jnp.concatenate([x[:H], x[H:]], axis=1)
    gpk_ref[...] = y.reshape(BO)

    def vox(row):
        g = (ptsT_ref[row, :] * RESOLUTION).astype(jnp.int32)
        return jnp.clip(g, 0, RESOLUTION - 1)

    gx = vox(0)
    gy = vox(1)
    gz = vox(2)
    t = ts_ref[...]
    i0 = jnp.clip((t * (NUM_FRAMES - 1)).astype(jnp.int32) + 1,
                  1, NUM_FRAMES - 1)
    left = (i0 - 1).astype(jnp.float32) * _INV63
    right = i0.astype(jnp.float32) * _INV63
    fidx = jnp.where((t - left) <= (right - t), i0 - 1, i0)
    u = (fidx << 12) | (gx << 6) | gy
    idx_ref[...] = (((u >> 13) << 19) | ((u & (H - 1)) << 7)
                    | (((u >> 12) & 1) << 6) | gz)


def _sc_body(gpad_hbm, idx_hbm, vals_hbm, idx_v0, idx_v1, vals_v0, vals_v1,
             sem_in0, sem_in1, sem_out0, sem_out1, sem_g):
    wid = lax.axis_index("s") * NC + lax.axis_index("c")
    base = wid * PPW
    sem_in = (sem_in0, sem_in1)
    sem_out = (sem_out0, sem_out1)
    idx_v = (idx_v0, idx_v1)
    vals_v = (vals_v0, vals_v1)

    def in_copy(c, b):
        return pltpu.make_async_copy(
            idx_hbm.at[pl.ds(base + c * CHUNK, CHUNK)], idx_v[b],
            sem_in[b])

    def out_copy(c, b):
        return pltpu.make_async_copy(
            vals_v[b], vals_hbm.at[pl.ds(base + c * CHUNK, CHUNK)],
            sem_out[b])

    in_copy(0, 0).start()
    for c in range(NCHUNK):
        b = c & 1
        in_copy(c, b).wait()
        if c + 1 < NCHUNK:
            in_copy(c + 1, 1 - b).start()
        if c >= 2:
            out_copy(c - 2, b).wait()
        pltpu.async_copy(gpad_hbm.at[idx_v[b]], vals_v[b], sem_g).wait()
        out_copy(c, b).start()
    out_copy(NCHUNK - 2, 0).wait()
    out_copy(NCHUNK - 1, 1).wait()


@jax.jit
def kernel(pts, ts, ts_keyframes, occ_val_grid):
    ptsT = pts.T
    g2 = occ_val_grid.reshape(G_ROWS, RESOLUTION)

    gpad, idx = pl.pallas_call(
        _tc_body,
        grid=(TC_STEPS,),
        in_specs=[
            pl.BlockSpec((BR, RESOLUTION), lambda i: (i, 0)),
            pl.BlockSpec((3, BP), lambda i: (0, i)),
            pl.BlockSpec((BP,), lambda i: (i,)),
        ],
        out_specs=[
            pl.BlockSpec((BO,), lambda i: (i,)),
            pl.BlockSpec((BP,), lambda i: (i,)),
        ],
        out_shape=[
            jax.ShapeDtypeStruct((G_ELEMS,), jnp.float32),
            jax.ShapeDtypeStruct((N,), jnp.int32),
        ],
        compiler_params=pltpu.CompilerParams(
            dimension_semantics=("arbitrary",),
        ),
    )(g2, ptsT, ts)

    mesh = plsc.VectorSubcoreMesh(core_axis_name="c", subcore_axis_name="s")
    fn = pl.kernel(
        _sc_body,
        mesh=mesh,
        compiler_params=pltpu.CompilerParams(needs_layout_passes=False),
        out_type=jax.ShapeDtypeStruct((N,), jnp.float32),
        scratch_types=[
            pltpu.VMEM((CHUNK,), jnp.int32),
            pltpu.VMEM((CHUNK,), jnp.int32),
            pltpu.VMEM((CHUNK,), jnp.float32),
            pltpu.VMEM((CHUNK,), jnp.float32),
            pltpu.SemaphoreType.DMA,
            pltpu.SemaphoreType.DMA,
            pltpu.SemaphoreType.DMA,
            pltpu.SemaphoreType.DMA,
            pltpu.SemaphoreType.DMA,
        ],
    )
    vals = fn(gpad, idx)
    return (vals, vals > OCC_THRE)
